# baseline (speedup 1.0000x reference)
import functools

import jax
import jax.numpy as jnp
from jax import lax
from jax.experimental import pallas as pl
from jax.experimental.pallas import tpu as pltpu
from jax.experimental.pallas import tpu_sc as plsc

B, N, DEG, F = 8, 2048, 16, 128
R = B * N
NC, NS, L = 2, 16, 16
NW = NC * NS
RPW = R // NW
CH = 8
NCHUNK = RPW // CH
FC = F // L


def _tree17(vals, op):
    while len(vals) > 1:
        nxt = [op(vals[i], vals[i + 1]) for i in range(0, len(vals) - 1, 2)]
        if len(vals) % 2:
            nxt.append(vals[-1])
        vals = nxt
    return vals[0]


def _reduce_chunk(gbuf, sbuf, obuf, is_max, j):
    op = jnp.maximum if is_max else jnp.add
    def crow(c, _):
        gb = c * DEG
        for fc in range(FC):
            sl = pl.ds(fc * L, L)
            vals = ([sbuf[j * CH + c, sl]]
                    + [gbuf[gb + d, sl] for d in range(DEG)])
            obuf[c, sl] = _tree17(vals, op)
        return 0
    lax.fori_loop(0, CH, crow, 0)


def _issue_gather(h_hbm, ebuf, j, gbuf, lsem):
    pltpu.async_copy(h_hbm.at[ebuf.at[j]], gbuf, lsem)


def _wait_gather(h_hbm, ebuf, j, gbuf, lsem):
    pltpu.make_async_copy(h_hbm.at[ebuf.at[j]], gbuf, lsem).wait()


def _gather_phase_body(h_hbm, eg_hbm, out_hbm,
                       ebuf, gbufA, gbufB, sbuf, obufA, obufB,
                       lsemA, lsemB, osemA, osemB, ssem, *, is_max):
    w = lax.axis_index("s") * NC + lax.axis_index("c")
    row0 = w * RPW
    self_cp = pltpu.async_copy(h_hbm.at[pl.ds(row0, RPW)], sbuf, ssem)
    pltpu.sync_copy(eg_hbm.at[pl.ds(w * NCHUNK, NCHUNK)], ebuf)
    _issue_gather(h_hbm, ebuf, 0, gbufA, lsemA)
    _issue_gather(h_hbm, ebuf, 1, gbufB, lsemB)
    self_cp.wait()

    def store(obuf, j, osem):
        pltpu.async_copy(obuf, out_hbm.at[pl.ds(row0 + j * CH, CH)], osem)

    def wait_store(obuf, j, osem):
        pltpu.make_async_copy(
            obuf, out_hbm.at[pl.ds(row0 + j * CH, CH)], osem).wait()

    _wait_gather(h_hbm, ebuf, 0, gbufA, lsemA)
    _reduce_chunk(gbufA, sbuf, obufA, is_max, 0)
    _issue_gather(h_hbm, ebuf, 2, gbufA, lsemA)
    store(obufA, 0, osemA)
    _wait_gather(h_hbm, ebuf, 1, gbufB, lsemB)
    _reduce_chunk(gbufB, sbuf, obufB, is_max, 1)
    _issue_gather(h_hbm, ebuf, 3, gbufB, lsemB)
    store(obufB, 1, osemB)

    def step(jj, _):
        j0 = jj * 2
        j1 = j0 + 1
        _wait_gather(h_hbm, ebuf, j0, gbufA, lsemA)
        wait_store(obufA, j0 - 2, osemA)
        _reduce_chunk(gbufA, sbuf, obufA, is_max, j0)
        _issue_gather(h_hbm, ebuf, j0 + 2, gbufA, lsemA)
        store(obufA, j0, osemA)
        _wait_gather(h_hbm, ebuf, j1, gbufB, lsemB)
        wait_store(obufB, j1 - 2, osemB)
        _reduce_chunk(gbufB, sbuf, obufB, is_max, j1)
        _issue_gather(h_hbm, ebuf, j1 + 2, gbufB, lsemB)
        store(obufB, j1, osemB)
        return 0

    lax.fori_loop(1, NCHUNK // 2 - 1, step, 0)
    j0 = NCHUNK - 2
    _wait_gather(h_hbm, ebuf, j0, gbufA, lsemA)
    wait_store(obufA, j0 - 2, osemA)
    _reduce_chunk(gbufA, sbuf, obufA, is_max, j0)
    store(obufA, j0, osemA)
    j1 = NCHUNK - 1
    _wait_gather(h_hbm, ebuf, j1, gbufB, lsemB)
    wait_store(obufB, j1 - 2, osemB)
    _reduce_chunk(gbufB, sbuf, obufB, is_max, j1)
    store(obufB, j1, osemB)
    wait_store(obufA, j0, osemA)
    wait_store(obufB, j1, osemB)


def _pool_sum_body(h_hbm, eg_hbm, out_hbm,
                   ebuf, gbufA, gbufB, sbuf, accv,
                   lsemA, lsemB, ssem):
    w = lax.axis_index("s") * NC + lax.axis_index("c")
    row0 = w * RPW
    self_cp = pltpu.async_copy(h_hbm.at[pl.ds(row0, RPW)], sbuf, ssem)
    pltpu.sync_copy(eg_hbm.at[pl.ds(w * NCHUNK, NCHUNK)], ebuf)
    zero = jnp.zeros((L,), jnp.float32)
    for fc in range(FC):
        accv[pl.ds(fc * L, L)] = zero
    _issue_gather(h_hbm, ebuf, 0, gbufA, lsemA)
    _issue_gather(h_hbm, ebuf, 1, gbufB, lsemB)
    self_cp.wait()

    def pool_acc(gbuf, j):
        def crow(c, _):
            gb = c * DEG
            for fc in range(FC):
                sl = pl.ds(fc * L, L)
                vals = ([sbuf[j * CH + c, sl]]
                        + [gbuf[gb + d, sl] for d in range(DEG)])
                accv[sl] = accv[sl] + _tree17(vals, jnp.maximum)
            return 0
        lax.fori_loop(0, CH, crow, 0)

    def step(jj, _):
        j0 = jj * 2
        j1 = j0 + 1
        _wait_gather(h_hbm, ebuf, j0, gbufA, lsemA)
        pool_acc(gbufA, j0)
        _issue_gather(h_hbm, ebuf, j0 + 2, gbufA, lsemA)
        _wait_gather(h_hbm, ebuf, j1, gbufB, lsemB)
        pool_acc(gbufB, j1)
        _issue_gather(h_hbm, ebuf, j1 + 2, gbufB, lsemB)
        return 0

    lax.fori_loop(0, NCHUNK // 2 - 1, step, 0)
    j0 = NCHUNK - 2
    _wait_gather(h_hbm, ebuf, j0, gbufA, lsemA)
    pool_acc(gbufA, j0)
    j1 = NCHUNK - 1
    _wait_gather(h_hbm, ebuf, j1, gbufB, lsemB)
    pool_acc(gbufB, j1)
    pltpu.sync_copy(accv, out_hbm.at[w])


_GATHER_SCRATCH = [
    pltpu.VMEM((NCHUNK, DEG * CH), jnp.int32),
    pltpu.VMEM((CH * DEG, F), jnp.float32),
    pltpu.VMEM((CH * DEG, F), jnp.float32),
    pltpu.VMEM((RPW, F), jnp.float32),
    pltpu.VMEM((CH, F), jnp.float32),
    pltpu.VMEM((CH, F), jnp.float32),
    pltpu.SemaphoreType.DMA,
    pltpu.SemaphoreType.DMA,
    pltpu.SemaphoreType.DMA,
    pltpu.SemaphoreType.DMA,
    pltpu.SemaphoreType.DMA,
]

@functools.cache
def _sc_kernels():
    mesh = plsc.VectorSubcoreMesh(
        core_axis_name="c", subcore_axis_name="s",
        num_cores=NC, num_subcores=NS)
    gather_sum = functools.partial(
        pl.kernel,
        out_type=jax.ShapeDtypeStruct((R, F), jnp.float32),
        mesh=mesh,
        scratch_types=_GATHER_SCRATCH,
    )(functools.partial(_gather_phase_body, is_max=False))
    gather_max = functools.partial(
        pl.kernel,
        out_type=jax.ShapeDtypeStruct((R, F), jnp.float32),
        mesh=mesh,
        scratch_types=_GATHER_SCRATCH,
    )(functools.partial(_gather_phase_body, is_max=True))
    pool_sum = functools.partial(
        pl.kernel,
        out_type=jax.ShapeDtypeStruct((NW, F), jnp.float32),
        mesh=mesh,
        scratch_types=[
            pltpu.VMEM((NCHUNK, DEG * CH), jnp.int32),
            pltpu.VMEM((CH * DEG, F), jnp.float32),
            pltpu.VMEM((CH * DEG, F), jnp.float32),
            pltpu.VMEM((RPW, F), jnp.float32),
            pltpu.VMEM((F,), jnp.float32),
            pltpu.SemaphoreType.DMA,
            pltpu.SemaphoreType.DMA,
            pltpu.SemaphoreType.DMA,
        ],
    )(_pool_sum_body)
    return gather_sum, gather_max, pool_sum


def _mm_relu_body(x_ref, w_ref, b_ref, o_ref):
    o_ref[...] = jnp.maximum(
        jnp.dot(x_ref[...], w_ref[...], preferred_element_type=jnp.float32)
        + b_ref[...], 0.0)


_MM_ROWS = 1024

_mm_relu = pl.pallas_call(
    _mm_relu_body,
    grid=(R // _MM_ROWS,),
    in_specs=[
        pl.BlockSpec((_MM_ROWS, F), lambda i: (i, 0)),
        pl.BlockSpec((F, F), lambda i: (0, 0)),
        pl.BlockSpec((1, F), lambda i: (0, 0)),
    ],
    out_specs=pl.BlockSpec((_MM_ROWS, F), lambda i: (i, 0)),
    out_shape=jax.ShapeDtypeStruct((R, F), jnp.float32),
)


def kernel(a, b, e, W1, b1, W2, b2):
    del b
    a2 = a.reshape(R, F)
    eg = (e.astype(jnp.int32)
          + (jnp.arange(B, dtype=jnp.int32) * N)[:, None, None])
    eg2d = eg.reshape(R * DEG // 128, 128)
    gather_sum, gather_max, pool_sum = _sc_kernels()
    s1 = gather_sum(a2, eg2d)
    h1 = _mm_relu(s1, W1, b1.reshape(1, F))
    p1 = gather_max(h1, eg2d)
    s2 = gather_sum(p1, eg2d)
    h2 = _mm_relu(s2, W2, b2.reshape(1, F))
    part = pool_sum(h2, eg2d)
    return part.reshape(B, NW // B, F).sum(axis=1)


# speedup vs baseline: 1.1075x; 1.1075x over previous
import functools

import jax
import jax.numpy as jnp
from jax import lax
from jax.experimental import pallas as pl
from jax.experimental.pallas import tpu as pltpu
from jax.experimental.pallas import tpu_sc as plsc

B, N, DEG, F = 8, 2048, 16, 128
R = B * N
NC, NS, L = 2, 16, 16
NW = NC * NS
RPW = R // NW
CH = 8
NCHUNK = RPW // CH
FC = F // L


def _tree17(vals, op):
    while len(vals) > 1:
        nxt = [op(vals[i], vals[i + 1]) for i in range(0, len(vals) - 1, 2)]
        if len(vals) % 2:
            nxt.append(vals[-1])
        vals = nxt
    return vals[0]


def _reduce_chunk(gbuf, sbuf, obuf, is_max, j):
    op = jnp.maximum if is_max else jnp.add

    @plsc.parallel_loop(0, CH, 1, unroll=2)
    def crow(c):
        gb = c * DEG
        for fc in range(FC):
            sl = pl.ds(fc * L, L)
            vals = ([sbuf[j * CH + c, sl]]
                    + [gbuf[gb + d, sl] for d in range(DEG)])
            obuf[c, sl] = _tree17(vals, op)


def _issue_gather(h_hbm, ebuf, j, gbuf, lsem):
    pltpu.async_copy(h_hbm.at[ebuf.at[j]], gbuf, lsem)


def _wait_gather(h_hbm, ebuf, j, gbuf, lsem):
    pltpu.make_async_copy(h_hbm.at[ebuf.at[j]], gbuf, lsem).wait()


def _gather_phase_body(h_hbm, eg_hbm, out_hbm,
                       ebuf, gbufA, gbufB, sbuf, obufA, obufB,
                       lsemA, lsemB, osemA, osemB, ssem, *, is_max):
    w = lax.axis_index("s") * NC + lax.axis_index("c")
    row0 = w * RPW
    self_cp = pltpu.async_copy(h_hbm.at[pl.ds(row0, RPW)], sbuf, ssem)
    pltpu.sync_copy(eg_hbm.at[pl.ds(w * NCHUNK, NCHUNK)], ebuf)
    _issue_gather(h_hbm, ebuf, 0, gbufA, lsemA)
    _issue_gather(h_hbm, ebuf, 1, gbufB, lsemB)
    self_cp.wait()

    def store(obuf, j, osem):
        pltpu.async_copy(obuf, out_hbm.at[pl.ds(row0 + j * CH, CH)], osem)

    def wait_store(obuf, j, osem):
        pltpu.make_async_copy(
            obuf, out_hbm.at[pl.ds(row0 + j * CH, CH)], osem).wait()

    _wait_gather(h_hbm, ebuf, 0, gbufA, lsemA)
    _reduce_chunk(gbufA, sbuf, obufA, is_max, 0)
    _issue_gather(h_hbm, ebuf, 2, gbufA, lsemA)
    store(obufA, 0, osemA)
    _wait_gather(h_hbm, ebuf, 1, gbufB, lsemB)
    _reduce_chunk(gbufB, sbuf, obufB, is_max, 1)
    _issue_gather(h_hbm, ebuf, 3, gbufB, lsemB)
    store(obufB, 1, osemB)

    def step(jj, _):
        j0 = jj * 2
        j1 = j0 + 1
        _wait_gather(h_hbm, ebuf, j0, gbufA, lsemA)
        wait_store(obufA, j0 - 2, osemA)
        _reduce_chunk(gbufA, sbuf, obufA, is_max, j0)
        _issue_gather(h_hbm, ebuf, j0 + 2, gbufA, lsemA)
        store(obufA, j0, osemA)
        _wait_gather(h_hbm, ebuf, j1, gbufB, lsemB)
        wait_store(obufB, j1 - 2, osemB)
        _reduce_chunk(gbufB, sbuf, obufB, is_max, j1)
        _issue_gather(h_hbm, ebuf, j1 + 2, gbufB, lsemB)
        store(obufB, j1, osemB)
        return 0

    lax.fori_loop(1, NCHUNK // 2 - 1, step, 0)
    j0 = NCHUNK - 2
    _wait_gather(h_hbm, ebuf, j0, gbufA, lsemA)
    wait_store(obufA, j0 - 2, osemA)
    _reduce_chunk(gbufA, sbuf, obufA, is_max, j0)
    store(obufA, j0, osemA)
    j1 = NCHUNK - 1
    _wait_gather(h_hbm, ebuf, j1, gbufB, lsemB)
    wait_store(obufB, j1 - 2, osemB)
    _reduce_chunk(gbufB, sbuf, obufB, is_max, j1)
    store(obufB, j1, osemB)
    wait_store(obufA, j0, osemA)
    wait_store(obufB, j1, osemB)


def _pool_sum_body(h_hbm, eg_hbm, out_hbm,
                   ebuf, gbufA, gbufB, sbuf, accv,
                   lsemA, lsemB, ssem):
    w = lax.axis_index("s") * NC + lax.axis_index("c")
    row0 = w * RPW
    self_cp = pltpu.async_copy(h_hbm.at[pl.ds(row0, RPW)], sbuf, ssem)
    pltpu.sync_copy(eg_hbm.at[pl.ds(w * NCHUNK, NCHUNK)], ebuf)
    zero = jnp.zeros((L,), jnp.float32)
    for fc in range(FC):
        accv[pl.ds(fc * L, L)] = zero
    _issue_gather(h_hbm, ebuf, 0, gbufA, lsemA)
    _issue_gather(h_hbm, ebuf, 1, gbufB, lsemB)
    self_cp.wait()

    def pool_acc(gbuf, j):
        accs = tuple(accv[pl.ds(fc * L, L)] for fc in range(FC))

        @plsc.parallel_loop(0, CH, 1, unroll=2, carry=accs)
        def crow(c, acc):
            gb = c * DEG
            out = []
            for fc in range(FC):
                sl = pl.ds(fc * L, L)
                vals = ([sbuf[j * CH + c, sl]]
                        + [gbuf[gb + d, sl] for d in range(DEG)])
                out.append(acc[fc] + _tree17(vals, jnp.maximum))
            return tuple(out)

        for fc in range(FC):
            accv[pl.ds(fc * L, L)] = crow[fc]

    def step(jj, _):
        j0 = jj * 2
        j1 = j0 + 1
        _wait_gather(h_hbm, ebuf, j0, gbufA, lsemA)
        pool_acc(gbufA, j0)
        _issue_gather(h_hbm, ebuf, j0 + 2, gbufA, lsemA)
        _wait_gather(h_hbm, ebuf, j1, gbufB, lsemB)
        pool_acc(gbufB, j1)
        _issue_gather(h_hbm, ebuf, j1 + 2, gbufB, lsemB)
        return 0

    lax.fori_loop(0, NCHUNK // 2 - 1, step, 0)
    j0 = NCHUNK - 2
    _wait_gather(h_hbm, ebuf, j0, gbufA, lsemA)
    pool_acc(gbufA, j0)
    j1 = NCHUNK - 1
    _wait_gather(h_hbm, ebuf, j1, gbufB, lsemB)
    pool_acc(gbufB, j1)
    pltpu.sync_copy(accv, out_hbm.at[w])


_GATHER_SCRATCH = [
    pltpu.VMEM((NCHUNK, DEG * CH), jnp.int32),
    pltpu.VMEM((CH * DEG, F), jnp.float32),
    pltpu.VMEM((CH * DEG, F), jnp.float32),
    pltpu.VMEM((RPW, F), jnp.float32),
    pltpu.VMEM((CH, F), jnp.float32),
    pltpu.VMEM((CH, F), jnp.float32),
    pltpu.SemaphoreType.DMA,
    pltpu.SemaphoreType.DMA,
    pltpu.SemaphoreType.DMA,
    pltpu.SemaphoreType.DMA,
    pltpu.SemaphoreType.DMA,
]

@functools.cache
def _sc_kernels():
    mesh = plsc.VectorSubcoreMesh(
        core_axis_name="c", subcore_axis_name="s",
        num_cores=NC, num_subcores=NS)
    gather_sum = functools.partial(
        pl.kernel,
        out_type=jax.ShapeDtypeStruct((R, F), jnp.float32),
        mesh=mesh,
        scratch_types=_GATHER_SCRATCH,
    )(functools.partial(_gather_phase_body, is_max=False))
    gather_max = functools.partial(
        pl.kernel,
        out_type=jax.ShapeDtypeStruct((R, F), jnp.float32),
        mesh=mesh,
        scratch_types=_GATHER_SCRATCH,
    )(functools.partial(_gather_phase_body, is_max=True))
    pool_sum = functools.partial(
        pl.kernel,
        out_type=jax.ShapeDtypeStruct((NW, F), jnp.float32),
        mesh=mesh,
        scratch_types=[
            pltpu.VMEM((NCHUNK, DEG * CH), jnp.int32),
            pltpu.VMEM((CH * DEG, F), jnp.float32),
            pltpu.VMEM((CH * DEG, F), jnp.float32),
            pltpu.VMEM((RPW, F), jnp.float32),
            pltpu.VMEM((F,), jnp.float32),
            pltpu.SemaphoreType.DMA,
            pltpu.SemaphoreType.DMA,
            pltpu.SemaphoreType.DMA,
        ],
    )(_pool_sum_body)
    return gather_sum, gather_max, pool_sum


def _mm_relu_body(x_ref, w_ref, b_ref, o_ref):
    o_ref[...] = jnp.maximum(
        jnp.dot(x_ref[...], w_ref[...], preferred_element_type=jnp.float32)
        + b_ref[...], 0.0)


_MM_ROWS = 1024

_mm_relu = pl.pallas_call(
    _mm_relu_body,
    grid=(R // _MM_ROWS,),
    in_specs=[
        pl.BlockSpec((_MM_ROWS, F), lambda i: (i, 0)),
        pl.BlockSpec((F, F), lambda i: (0, 0)),
        pl.BlockSpec((1, F), lambda i: (0, 0)),
    ],
    out_specs=pl.BlockSpec((_MM_ROWS, F), lambda i: (i, 0)),
    out_shape=jax.ShapeDtypeStruct((R, F), jnp.float32),
)


def kernel(a, b, e, W1, b1, W2, b2):
    del b
    a2 = a.reshape(R, F)
    eg = (e.astype(jnp.int32)
          + (jnp.arange(B, dtype=jnp.int32) * N)[:, None, None])
    eg2d = eg.reshape(R * DEG // 128, 128)
    gather_sum, gather_max, pool_sum = _sc_kernels()
    s1 = gather_sum(a2, eg2d)
    h1 = _mm_relu(s1, W1, b1.reshape(1, F))
    p1 = gather_max(h1, eg2d)
    s2 = gather_sum(p1, eg2d)
    h2 = _mm_relu(s2, W2, b2.reshape(1, F))
    part = pool_sum(h2, eg2d)
    return part.reshape(B, NW // B, F).sum(axis=1)


# speedup vs baseline: 1.3336x; 1.2042x over previous
import functools

import jax
import jax.numpy as jnp
from jax import lax
from jax.experimental import pallas as pl
from jax.experimental.pallas import tpu as pltpu
from jax.experimental.pallas import tpu_sc as plsc

B, N, DEG, F = 8, 2048, 16, 128
R = B * N
NC, NS, L = 2, 16, 16
NW = NC * NS
RPW = R // NW
CH = 8
NCHUNK = RPW // CH
FC = F // L
NSLOT = 4
NITER = NCHUNK // NSLOT


def _tree17(vals, op):
    while len(vals) > 1:
        nxt = [op(vals[i], vals[i + 1]) for i in range(0, len(vals) - 1, 2)]
        if len(vals) % 2:
            nxt.append(vals[-1])
        vals = nxt
    return vals[0]


def _reduce_chunk(gbuf, sbuf, obuf, is_max):
    op = jnp.maximum if is_max else jnp.add

    @plsc.parallel_loop(0, CH, 1, unroll=2)
    def crow(c):
        gb = c * DEG
        for fc in range(FC):
            sl = pl.ds(fc * L, L)
            vals = [sbuf[c, sl]] + [gbuf[gb + d, sl] for d in range(DEG)]
            obuf[c, sl] = _tree17(vals, op)


def _issue_loads(h_hbm, ebuf, j, gbuf, sbuf, lsem, row0):
    pltpu.async_copy(h_hbm.at[ebuf.at[j]], gbuf, lsem)
    pltpu.async_copy(h_hbm.at[pl.ds(row0 + j * CH, CH)], sbuf, lsem)


def _wait_loads(h_hbm, ebuf, j, gbuf, sbuf, lsem, row0):
    pltpu.make_async_copy(h_hbm.at[ebuf.at[j]], gbuf, lsem).wait()
    pltpu.make_async_copy(h_hbm.at[pl.ds(row0 + j * CH, CH)], sbuf, lsem).wait()


def _gather_phase_body(h_hbm, eg_hbm, out_hbm, ebuf, *bufs, is_max):
    gb = bufs[0:NSLOT]
    sb = bufs[NSLOT:2 * NSLOT]
    ob = bufs[2 * NSLOT:3 * NSLOT]
    ls = bufs[3 * NSLOT:4 * NSLOT]
    os_ = bufs[4 * NSLOT:5 * NSLOT]
    w = lax.axis_index("s") * NC + lax.axis_index("c")
    row0 = w * RPW
    pltpu.sync_copy(eg_hbm.at[pl.ds(w * NCHUNK, NCHUNK)], ebuf)
    for k in range(NSLOT):
        _issue_loads(h_hbm, ebuf, k, gb[k], sb[k], ls[k], row0)

    def store(k, j):
        pltpu.async_copy(ob[k], out_hbm.at[pl.ds(row0 + j * CH, CH)], os_[k])

    def wait_store(k, j):
        pltpu.make_async_copy(
            ob[k], out_hbm.at[pl.ds(row0 + j * CH, CH)], os_[k]).wait()

    for k in range(NSLOT):
        _wait_loads(h_hbm, ebuf, k, gb[k], sb[k], ls[k], row0)
        _reduce_chunk(gb[k], sb[k], ob[k], is_max)
        _issue_loads(h_hbm, ebuf, k + NSLOT, gb[k], sb[k], ls[k], row0)
        store(k, k)

    def step(jj, _):
        j0 = jj * NSLOT
        for k in range(NSLOT):
            j = j0 + k
            _wait_loads(h_hbm, ebuf, j, gb[k], sb[k], ls[k], row0)
            wait_store(k, j - NSLOT)
            _reduce_chunk(gb[k], sb[k], ob[k], is_max)
            _issue_loads(h_hbm, ebuf, j + NSLOT, gb[k], sb[k], ls[k], row0)
            store(k, j)
        return 0

    lax.fori_loop(1, NITER - 1, step, 0)
    j0 = NCHUNK - NSLOT
    for k in range(NSLOT):
        j = j0 + k
        _wait_loads(h_hbm, ebuf, j, gb[k], sb[k], ls[k], row0)
        wait_store(k, j - NSLOT)
        _reduce_chunk(gb[k], sb[k], ob[k], is_max)
        store(k, j)
    for k in range(NSLOT):
        wait_store(k, j0 + k)


def _pool_sum_body(h_hbm, eg_hbm, out_hbm, ebuf, *bufs):
    gb = bufs[0:NSLOT]
    sb = bufs[NSLOT:2 * NSLOT]
    accv = bufs[2 * NSLOT]
    ls = bufs[2 * NSLOT + 1:]
    w = lax.axis_index("s") * NC + lax.axis_index("c")
    row0 = w * RPW
    pltpu.sync_copy(eg_hbm.at[pl.ds(w * NCHUNK, NCHUNK)], ebuf)
    zero = jnp.zeros((L,), jnp.float32)
    for fc in range(FC):
        accv[pl.ds(fc * L, L)] = zero
    for k in range(NSLOT):
        _issue_loads(h_hbm, ebuf, k, gb[k], sb[k], ls[k], row0)

    def pool_acc(gbuf, sbuf):
        accs = tuple(accv[pl.ds(fc * L, L)] for fc in range(FC))

        @plsc.parallel_loop(0, CH, 1, unroll=2, carry=accs)
        def crow(c, acc):
            g0 = c * DEG
            out = []
            for fc in range(FC):
                sl = pl.ds(fc * L, L)
                vals = [sbuf[c, sl]] + [gbuf[g0 + d, sl] for d in range(DEG)]
                out.append(acc[fc] + _tree17(vals, jnp.maximum))
            return tuple(out)

        for fc in range(FC):
            accv[pl.ds(fc * L, L)] = crow[fc]

    def step(jj, _):
        j0 = jj * NSLOT
        for k in range(NSLOT):
            j = j0 + k
            _wait_loads(h_hbm, ebuf, j, gb[k], sb[k], ls[k], row0)
            pool_acc(gb[k], sb[k])
            _issue_loads(h_hbm, ebuf, j + NSLOT, gb[k], sb[k], ls[k], row0)
        return 0

    lax.fori_loop(0, NITER - 1, step, 0)
    j0 = NCHUNK - NSLOT
    for k in range(NSLOT):
        _wait_loads(h_hbm, ebuf, j0 + k, gb[k], sb[k], ls[k], row0)
        pool_acc(gb[k], sb[k])
    pltpu.sync_copy(accv, out_hbm.at[w])


_GATHER_SCRATCH = (
    [pltpu.VMEM((NCHUNK, DEG * CH), jnp.int32)]
    + [pltpu.VMEM((CH * DEG, F), jnp.float32) for _ in range(NSLOT)]
    + [pltpu.VMEM((CH, F), jnp.float32) for _ in range(NSLOT)]
    + [pltpu.VMEM((CH, F), jnp.float32) for _ in range(NSLOT)]
    + [pltpu.SemaphoreType.DMA for _ in range(2 * NSLOT)]
)

_POOL_SCRATCH = (
    [pltpu.VMEM((NCHUNK, DEG * CH), jnp.int32)]
    + [pltpu.VMEM((CH * DEG, F), jnp.float32) for _ in range(NSLOT)]
    + [pltpu.VMEM((CH, F), jnp.float32) for _ in range(NSLOT)]
    + [pltpu.VMEM((F,), jnp.float32)]
    + [pltpu.SemaphoreType.DMA for _ in range(NSLOT)]
)


@functools.cache
def _sc_kernels():
    mesh = plsc.VectorSubcoreMesh(
        core_axis_name="c", subcore_axis_name="s",
        num_cores=NC, num_subcores=NS)
    gather_sum = functools.partial(
        pl.kernel,
        out_type=jax.ShapeDtypeStruct((R, F), jnp.float32),
        mesh=mesh,
        scratch_types=_GATHER_SCRATCH,
    )(functools.partial(_gather_phase_body, is_max=False))
    gather_max = functools.partial(
        pl.kernel,
        out_type=jax.ShapeDtypeStruct((R, F), jnp.float32),
        mesh=mesh,
        scratch_types=_GATHER_SCRATCH,
    )(functools.partial(_gather_phase_body, is_max=True))
    pool_sum = functools.partial(
        pl.kernel,
        out_type=jax.ShapeDtypeStruct((NW, F), jnp.float32),
        mesh=mesh,
        scratch_types=_POOL_SCRATCH,
    )(_pool_sum_body)
    return gather_sum, gather_max, pool_sum


def _mm_relu_body(x_ref, w_ref, b_ref, o_ref):
    o_ref[...] = jnp.maximum(
        jnp.dot(x_ref[...], w_ref[...], preferred_element_type=jnp.float32)
        + b_ref[...], 0.0)


_MM_ROWS = 1024

_mm_relu = pl.pallas_call(
    _mm_relu_body,
    grid=(R // _MM_ROWS,),
    in_specs=[
        pl.BlockSpec((_MM_ROWS, F), lambda i: (i, 0)),
        pl.BlockSpec((F, F), lambda i: (0, 0)),
        pl.BlockSpec((1, F), lambda i: (0, 0)),
    ],
    out_specs=pl.BlockSpec((_MM_ROWS, F), lambda i: (i, 0)),
    out_shape=jax.ShapeDtypeStruct((R, F), jnp.float32),
)


def kernel(a, b, e, W1, b1, W2, b2):
    del b
    a2 = a.reshape(R, F)
    eg = (e.astype(jnp.int32)
          + (jnp.arange(B, dtype=jnp.int32) * N)[:, None, None])
    eg2d = eg.reshape(R * DEG // 128, 128)
    gather_sum, gather_max, pool_sum = _sc_kernels()
    s1 = gather_sum(a2, eg2d)
    h1 = _mm_relu(s1, W1, b1.reshape(1, F))
    p1 = gather_max(h1, eg2d)
    s2 = gather_sum(p1, eg2d)
    h2 = _mm_relu(s2, W2, b2.reshape(1, F))
    part = pool_sum(h2, eg2d)
    return part.reshape(B, NW // B, F).sum(axis=1)
